# Initial kernel scaffold; baseline (speedup 1.0000x reference)
import jax
import jax.numpy as jnp
from jax.experimental import pallas as pl


def kernel(domain_strings, x, raw_weights):
    raise NotImplementedError("write your pallas kernel here")


# speedup vs baseline: 1.0972x; 1.0972x over previous
import functools

import jax
import jax.numpy as jnp
from jax import lax
from jax.experimental import pallas as pl
from jax.experimental.pallas import tpu as pltpu
from jax.experimental.pallas import tpu_sc as plsc

BATCH = 16384
NUM_DOMAINS = 26
TABLE_PAD = 32
NUM_CORES = 1
NUM_SUBCORES = 16
NUM_WORKERS = NUM_CORES * NUM_SUBCORES
CHUNK = BATCH // NUM_WORKERS
LANES = 16


def _softplus(w):
    t = jnp.exp(-jnp.abs(w))
    s = t / (2.0 + t)
    s2 = s * s
    log1p = 2.0 * s * (1.0 + s2 * (1.0 / 3.0 + s2 * (0.2 + s2 * (1.0 / 7.0))))
    return jnp.maximum(w, 0.0) + log1p


@functools.partial(
    pl.kernel,
    mesh=plsc.VectorSubcoreMesh(core_axis_name="c", subcore_axis_name="s",
                                num_cores=NUM_CORES,
                                num_subcores=NUM_SUBCORES),
    out_type=jax.ShapeDtypeStruct((BATCH,), jnp.float32),
    compiler_params=pltpu.CompilerParams(needs_layout_passes=False),
    scratch_types=[
        pltpu.VMEM((TABLE_PAD,), jnp.float32),
        pltpu.VMEM((TABLE_PAD,), jnp.float32),
        pltpu.VMEM((CHUNK,), jnp.int32),
        pltpu.VMEM((CHUNK,), jnp.float32),
        pltpu.VMEM((CHUNK,), jnp.float32),
        pltpu.SemaphoreType.DMA,
        pltpu.SemaphoreType.DMA,
        pltpu.SemaphoreType.DMA,
    ],
)
def _domain_scale(idx_hbm, x_hbm, w_hbm, out_hbm,
                  w_v, table_v, idx_v, x_v, out_v, sem_i, sem_x, sem_o):
    wid = lax.axis_index("s") * NUM_CORES + lax.axis_index("c")
    base = wid * CHUNK

    cp_w = pltpu.async_copy(w_hbm, w_v.at[pl.ds(0, NUM_DOMAINS)], sem_o)
    cp_i = pltpu.async_copy(idx_hbm.at[pl.ds(base, CHUNK)], idx_v, sem_i)
    cp_x = pltpu.async_copy(x_hbm.at[pl.ds(base, CHUNK)], x_v, sem_x)
    cp_w.wait()
    sp = []
    for j in range(TABLE_PAD // LANES):
        w = w_v[pl.ds(j * LANES, LANES)]
        sp.append(_softplus(w))
    lane = lax.iota(jnp.int32, LANES)
    masked = sp[0] + jnp.where(lane < NUM_DOMAINS - LANES, sp[1], 0.0)
    total = jnp.sum(masked)
    scale = jnp.float32(NUM_DOMAINS) / lax.broadcast_in_dim(
        total, (LANES,), ())
    for j in range(TABLE_PAD // LANES):
        table_v[pl.ds(j * LANES, LANES)] = sp[j] * scale

    cp_i.wait()
    cp_x.wait()

    @plsc.parallel_loop(0, CHUNK, step=LANES, unroll=8)
    def _gather(i):
        sl = pl.ds(i, LANES)
        wv = plsc.load_gather(table_v, [idx_v[sl]])
        out_v[sl] = x_v[sl] * wv

    pltpu.sync_copy(out_v, out_hbm.at[pl.ds(base, CHUNK)])


def kernel(domain_strings, x, raw_weights):
    idx = domain_strings.astype(jnp.int32)
    xf = x.reshape(BATCH).astype(jnp.float32)
    out = _domain_scale(idx, xf, raw_weights.astype(jnp.float32))
    return out.reshape(BATCH, 1)
